# Initial kernel scaffold; baseline (speedup 1.0000x reference)
import jax
import jax.numpy as jnp
from jax.experimental import pallas as pl


def kernel(x, token_table, pos_table):
    raise NotImplementedError("write your pallas kernel here")


# speedup vs baseline: 3.3456x; 3.3456x over previous
import functools

import jax
import jax.numpy as jnp
from jax import lax
from jax.experimental import pallas as pl
from jax.experimental.pallas import tpu as pltpu
from jax.experimental.pallas import tpu_sc as plsc

D = 64
L = 200
NC, NS = 2, 16
NW = NC * NS

BATCH = 4096
N = BATCH * L
SEQ_PW = BATCH // NW
SEQ_PER_CHUNK = 2
CHUNK = SEQ_PER_CHUNK * L
NCHUNKS = SEQ_PW // SEQ_PER_CHUNK
ROWS_PW = SEQ_PW * L


@functools.cache
def _embed_kernel():
    mesh = plsc.VectorSubcoreMesh(core_axis_name="c", subcore_axis_name="s")

    @functools.partial(
        pl.kernel,
        mesh=mesh,
        compiler_params=pltpu.CompilerParams(use_tc_tiling_on_sc=False),
        out_type=jax.ShapeDtypeStruct((N, D), jnp.float32),
        scratch_types=[
            pltpu.VMEM((CHUNK,), jnp.int32),
            pltpu.VMEM((CHUNK, D), jnp.float32),
            pltpu.VMEM((L, D), jnp.float32),
            pltpu.SemaphoreType.DMA,
        ],
    )
    def body(x_hbm, tok_hbm, pos_hbm, out_hbm, idx_v, rows_v, pos_v, sem):
        wid = lax.axis_index("s") * NC + lax.axis_index("c")
        base0 = wid * ROWS_PW
        pltpu.sync_copy(pos_hbm, pos_v)

        def chunk_body(c, carry):
            base = base0 + c * CHUNK
            pltpu.sync_copy(x_hbm.at[pl.ds(base, CHUNK)], idx_v)
            pltpu.async_copy(tok_hbm.at[idx_v], rows_v, sem).wait()

            def row_body(j, carry2):
                for s in range(SEQ_PER_CHUNK):
                    for g in range(D // 16):
                        sl = pl.ds(g * 16, 16)
                        plsc.addupdate(rows_v.at[s * L + j, sl], pos_v[j, sl])
                return carry2

            lax.fori_loop(0, L, row_body, 0)
            pltpu.sync_copy(rows_v, out_hbm.at[pl.ds(base, CHUNK)])
            return carry

        lax.fori_loop(0, NCHUNKS, chunk_body, 0)

    return body


def kernel(x, token_table, pos_table):
    xf = x.reshape(-1).astype(jnp.int32)
    out = _embed_kernel()(xf, token_table, pos_table)
    return out.reshape(x.shape[0], x.shape[1], D)
